# baseline (speedup 1.0000x reference)
import functools

import jax
import jax.numpy as jnp
from jax import lax
from jax.experimental import pallas as pl
from jax.experimental.pallas import tpu as pltpu
from jax.experimental.pallas import tpu_sc as plsc

NUM_COMPONENTS = 32
BATCH_U = 4096
BATCH_I = 4096

_info = plsc.get_sparse_core_info()
_NC = _info.num_cores
_NS = _info.num_subcores
_NW = _NC * _NS
_NBUF = 16

_sc_mesh = plsc.VectorSubcoreMesh(core_axis_name="c", subcore_axis_name="s")


def _gather_tables(table_specs, ring_v, sems):
    c_iota = lax.iota(jnp.int32, 16)

    for emb_t_hbm, idx_v, n_ids, cols_v in table_specs:
        n_grp = n_ids // _NBUF

        def fire(g, b, idx_v=idx_v, emb_t_hbm=emb_t_hbm):
            vec = idx_v[pl.ds(g * _NBUF, _NBUF)]
            rid = vec[b]
            toff = pl.multiple_of((rid // 128) * 128, 128)
            pltpu.async_copy(emb_t_hbm.at[:, pl.ds(toff, 128)],
                             ring_v.at[b], sems[b])

        def extract(g, b, idx_v=idx_v, cols_v=cols_v):
            vec = idx_v[pl.ds(g * _NBUF, _NBUF)]
            lane = jnp.broadcast_to(vec[b] % 128, (16,))
            row = jnp.broadcast_to(g * _NBUF + b, (16,))
            lo = plsc.load_gather(ring_v.at[b], [c_iota, lane])
            hi = plsc.load_gather(ring_v.at[b], [c_iota + 16, lane])
            plsc.store_scatter(cols_v, [row, c_iota], lo)
            plsc.store_scatter(cols_v, [row, c_iota + 16], hi)

        def group(g, carry, emb_t_hbm=emb_t_hbm, fire=fire, extract=extract):
            for b in range(_NBUF):
                @pl.when(g > 0)
                def _drain():
                    pltpu.make_async_copy(
                        emb_t_hbm.at[:, pl.ds(0, 128)], ring_v.at[b],
                        sems[b]).wait()
                    extract(g - 1, b)
                fire(g, b)
            return carry

        lax.fori_loop(0, n_grp, group, 0)
        for b in range(_NBUF):
            pltpu.make_async_copy(emb_t_hbm.at[:, pl.ds(0, 128)],
                                  ring_v.at[b], sems[b]).wait()
            extract(n_grp - 1, b)


_IDS_I = BATCH_I // _NW
_IDS_U = BATCH_U // _NW


@functools.partial(
    pl.kernel,
    out_type=[
        jax.ShapeDtypeStruct((BATCH_I, NUM_COMPONENTS), jnp.float32),
        jax.ShapeDtypeStruct((BATCH_U, NUM_COMPONENTS), jnp.float32),
    ],
    mesh=_sc_mesh,
    scratch_types=[
        pltpu.VMEM((_IDS_I,), jnp.int32),
        pltpu.VMEM((_IDS_U,), jnp.int32),
        pltpu.VMEM((_NBUF, NUM_COMPONENTS, 128), jnp.float32),
        pltpu.VMEM((_IDS_I, NUM_COMPONENTS), jnp.float32),
        pltpu.VMEM((_IDS_U, NUM_COMPONENTS), jnp.float32),
    ] + [pltpu.SemaphoreType.DMA] * _NBUF,
    compiler_params=pltpu.CompilerParams(needs_layout_passes=False),
)
def _sc_gather(item_idx_hbm, user_idx_hbm, item_emb_t_hbm, user_emb_t_hbm,
               hi_hbm, wu_hbm, iidx_v, uidx_v, ring_v, irows_v, urows_v,
               *sems):
    wid = lax.axis_index("s") * _NC + lax.axis_index("c")
    ibase = wid * _IDS_I
    ubase = wid * _IDS_U
    pltpu.sync_copy(item_idx_hbm.at[pl.ds(ibase, _IDS_I)], iidx_v)
    pltpu.sync_copy(user_idx_hbm.at[pl.ds(ubase, _IDS_U)], uidx_v)
    _gather_tables(
        [(item_emb_t_hbm, iidx_v, _IDS_I, irows_v),
         (user_emb_t_hbm, uidx_v, _IDS_U, urows_v)], ring_v, sems)
    pltpu.sync_copy(irows_v, hi_hbm.at[pl.ds(ibase, _IDS_I)])
    pltpu.sync_copy(urows_v, wu_hbm.at[pl.ds(ubase, _IDS_U)])


_BLK_U = 1024


def _mm_body(wu_ref, hi_ref, out_ref):
    out_ref[...] = lax.dot_general(
        wu_ref[...], hi_ref[...],
        dimension_numbers=(((1,), (1,)), ((), ())),
        preferred_element_type=jnp.float32,
    )


_matmul = pl.pallas_call(
    _mm_body,
    grid=(BATCH_U // _BLK_U,),
    in_specs=[
        pl.BlockSpec((_BLK_U, NUM_COMPONENTS), lambda i: (i, 0)),
        pl.BlockSpec((BATCH_I, NUM_COMPONENTS), lambda i: (0, 0)),
    ],
    out_specs=pl.BlockSpec((_BLK_U, BATCH_I), lambda i: (i, 0)),
    out_shape=jax.ShapeDtypeStruct((BATCH_U, BATCH_I), jnp.float32),
)


@jax.jit
def kernel(user_tensor, item_tensor, user_embedding, item_embedding):
    ue_t = user_embedding.T
    ie_t = item_embedding.T
    hi, wu = _sc_gather(item_tensor, user_tensor, ie_t, ue_t)
    return _matmul(wu, hi)


# speedup vs baseline: 1.0164x; 1.0164x over previous
import functools

import jax
import jax.numpy as jnp
from jax import lax
from jax.experimental import pallas as pl
from jax.experimental.pallas import tpu as pltpu
from jax.experimental.pallas import tpu_sc as plsc

NUM_COMPONENTS = 32
BATCH_U = 4096
BATCH_I = 4096

_info = plsc.get_sparse_core_info()
_NC = _info.num_cores
_NS = _info.num_subcores
_NW = _NC * _NS
_NBUF = 16

_sc_mesh = plsc.VectorSubcoreMesh(core_axis_name="c", subcore_axis_name="s")


def _gather_tables(table_specs, ring_v, sems):
    c_iota = lax.iota(jnp.int32, 16)

    for emb_t_hbm, idx_v, n_ids, cols_v in table_specs:
        n_grp = n_ids // _NBUF

        def fire(g, b, idx_v=idx_v, emb_t_hbm=emb_t_hbm):
            vec = idx_v[pl.ds(g * _NBUF, _NBUF)]
            rid = vec[b]
            toff = pl.multiple_of((rid // 128) * 128, 128)
            pltpu.async_copy(emb_t_hbm.at[:, pl.ds(toff, 128)],
                             ring_v.at[b], sems[b])

        def extract(g, b, idx_v=idx_v, cols_v=cols_v):
            vec = idx_v[pl.ds(g * _NBUF, _NBUF)]
            lane = jnp.broadcast_to(vec[b] % 128, (16,))
            row = jnp.broadcast_to(g * _NBUF + b, (16,))
            lo = plsc.load_gather(ring_v.at[b], [c_iota, lane])
            hi = plsc.load_gather(ring_v.at[b], [c_iota + 16, lane])
            plsc.store_scatter(cols_v, [row, c_iota], lo)
            plsc.store_scatter(cols_v, [row, c_iota + 16], hi)

        def group(g, carry, emb_t_hbm=emb_t_hbm, fire=fire, extract=extract):
            for b in range(_NBUF):
                @pl.when(g > 0)
                def _drain():
                    pltpu.make_async_copy(
                        emb_t_hbm.at[:, pl.ds(0, 128)], ring_v.at[b],
                        sems[b]).wait()
                    extract(g - 1, b)
                fire(g, b)
            return carry

        lax.fori_loop(0, n_grp, group, 0)
        for b in range(_NBUF):
            pltpu.make_async_copy(emb_t_hbm.at[:, pl.ds(0, 128)],
                                  ring_v.at[b], sems[b]).wait()
            extract(n_grp - 1, b)


_IDS_I = BATCH_I // _NW
_IDS_U = BATCH_U // _NW


@functools.partial(
    pl.kernel,
    out_type=[
        jax.ShapeDtypeStruct((BATCH_I, NUM_COMPONENTS), jnp.float32),
        jax.ShapeDtypeStruct((BATCH_U, NUM_COMPONENTS), jnp.float32),
    ],
    mesh=_sc_mesh,
    scratch_types=[
        pltpu.VMEM((_IDS_I,), jnp.int32),
        pltpu.VMEM((_IDS_U,), jnp.int32),
        pltpu.VMEM((_NBUF, NUM_COMPONENTS, 128), jnp.float32),
        pltpu.VMEM((_IDS_I, NUM_COMPONENTS), jnp.float32),
        pltpu.VMEM((_IDS_U, NUM_COMPONENTS), jnp.float32),
    ] + [pltpu.SemaphoreType.DMA] * _NBUF,
    compiler_params=pltpu.CompilerParams(needs_layout_passes=False),
)
def _sc_gather(item_idx_hbm, user_idx_hbm, item_emb_t_hbm, user_emb_t_hbm,
               hi_hbm, wu_hbm, iidx_v, uidx_v, ring_v, irows_v, urows_v,
               *sems):
    wid = lax.axis_index("s") * _NC + lax.axis_index("c")
    ibase = wid * _IDS_I
    ubase = wid * _IDS_U
    pltpu.sync_copy(item_idx_hbm.at[pl.ds(ibase, _IDS_I)], iidx_v)
    pltpu.sync_copy(user_idx_hbm.at[pl.ds(ubase, _IDS_U)], uidx_v)
    _gather_tables(
        [(item_emb_t_hbm, iidx_v, _IDS_I, irows_v),
         (user_emb_t_hbm, uidx_v, _IDS_U, urows_v)], ring_v, sems)
    pltpu.sync_copy(irows_v, hi_hbm.at[pl.ds(ibase, _IDS_I)])
    pltpu.sync_copy(urows_v, wu_hbm.at[pl.ds(ubase, _IDS_U)])


_BLK_U = 512


def _mm_body(wu_ref, hi_ref, out_ref):
    out_ref[...] = lax.dot_general(
        wu_ref[...], hi_ref[...],
        dimension_numbers=(((1,), (1,)), ((), ())),
        preferred_element_type=jnp.float32,
    )


_matmul = pl.pallas_call(
    _mm_body,
    grid=(BATCH_U // _BLK_U,),
    in_specs=[
        pl.BlockSpec((_BLK_U, NUM_COMPONENTS), lambda i: (i, 0)),
        pl.BlockSpec((BATCH_I, NUM_COMPONENTS), lambda i: (0, 0)),
    ],
    out_specs=pl.BlockSpec((_BLK_U, BATCH_I), lambda i: (i, 0)),
    out_shape=jax.ShapeDtypeStruct((BATCH_U, BATCH_I), jnp.float32),
)


@jax.jit
def kernel(user_tensor, item_tensor, user_embedding, item_embedding):
    ue_t = user_embedding.T
    ie_t = item_embedding.T
    hi, wu = _sc_gather(item_tensor, user_tensor, ie_t, ue_t)
    return _matmul(wu, hi)


# speedup vs baseline: 1.0275x; 1.0110x over previous
import functools

import jax
import jax.numpy as jnp
from jax import lax
from jax.experimental import pallas as pl
from jax.experimental.pallas import tpu as pltpu
from jax.experimental.pallas import tpu_sc as plsc

NUM_COMPONENTS = 32
BATCH_U = 4096
BATCH_I = 4096

_info = plsc.get_sparse_core_info()
_NC = _info.num_cores
_NS = _info.num_subcores
_NW = _NC * _NS
_NBUF = 16

_sc_mesh = plsc.VectorSubcoreMesh(core_axis_name="c", subcore_axis_name="s")


def _gather_tables(table_specs, ring_v, sems):
    c_iota = lax.iota(jnp.int32, 16)

    for emb_t_hbm, idx_v, n_ids, cols_v in table_specs:
        n_grp = n_ids // _NBUF

        def fire(g, b, idx_v=idx_v, emb_t_hbm=emb_t_hbm):
            vec = idx_v[pl.ds(g * _NBUF, _NBUF)]
            rid = vec[b]
            toff = pl.multiple_of((rid // 128) * 128, 128)
            pltpu.async_copy(emb_t_hbm.at[:, pl.ds(toff, 128)],
                             ring_v.at[b], sems[b])

        def extract(g, b, idx_v=idx_v, cols_v=cols_v):
            vec = idx_v[pl.ds(g * _NBUF, _NBUF)]
            lane = jnp.broadcast_to(vec[b] % 128, (16,))
            col = jnp.broadcast_to(g * _NBUF + b, (16,))
            lo = plsc.load_gather(ring_v.at[b], [c_iota, lane])
            hi = plsc.load_gather(ring_v.at[b], [c_iota + 16, lane])
            plsc.store_scatter(cols_v, [c_iota, col], lo)
            plsc.store_scatter(cols_v, [c_iota + 16, col], hi)

        def group(g, carry, emb_t_hbm=emb_t_hbm, fire=fire, extract=extract):
            for b in range(_NBUF):
                @pl.when(g > 0)
                def _drain():
                    pltpu.make_async_copy(
                        emb_t_hbm.at[:, pl.ds(0, 128)], ring_v.at[b],
                        sems[b]).wait()
                    extract(g - 1, b)
                fire(g, b)
            return carry

        lax.fori_loop(0, n_grp, group, 0)
        for b in range(_NBUF):
            pltpu.make_async_copy(emb_t_hbm.at[:, pl.ds(0, 128)],
                                  ring_v.at[b], sems[b]).wait()
            extract(n_grp - 1, b)


_IDS_I = BATCH_I // _NW
_IDS_U = BATCH_U // _NW


@functools.partial(
    pl.kernel,
    out_type=[
        jax.ShapeDtypeStruct((NUM_COMPONENTS, BATCH_I), jnp.float32),
        jax.ShapeDtypeStruct((NUM_COMPONENTS, BATCH_U), jnp.float32),
    ],
    mesh=_sc_mesh,
    scratch_types=[
        pltpu.VMEM((_IDS_I,), jnp.int32),
        pltpu.VMEM((_IDS_U,), jnp.int32),
        pltpu.VMEM((_NBUF, NUM_COMPONENTS, 128), jnp.float32),
        pltpu.VMEM((NUM_COMPONENTS, _IDS_I), jnp.float32),
        pltpu.VMEM((NUM_COMPONENTS, _IDS_U), jnp.float32),
    ] + [pltpu.SemaphoreType.DMA] * _NBUF,
    compiler_params=pltpu.CompilerParams(needs_layout_passes=False),
)
def _sc_gather(item_idx_hbm, user_idx_hbm, item_emb_t_hbm, user_emb_t_hbm,
               hi_t_hbm, wu_t_hbm, iidx_v, uidx_v, ring_v, icols_v, ucols_v,
               *sems):
    wid = lax.axis_index("s") * _NC + lax.axis_index("c")
    ibase = wid * _IDS_I
    ubase = wid * _IDS_U
    pltpu.sync_copy(item_idx_hbm.at[pl.ds(ibase, _IDS_I)], iidx_v)
    pltpu.sync_copy(user_idx_hbm.at[pl.ds(ubase, _IDS_U)], uidx_v)
    _gather_tables(
        [(item_emb_t_hbm, iidx_v, _IDS_I, icols_v),
         (user_emb_t_hbm, uidx_v, _IDS_U, ucols_v)], ring_v, sems)
    pltpu.sync_copy(icols_v, hi_t_hbm.at[:, pl.ds(ibase, _IDS_I)])
    pltpu.sync_copy(ucols_v, wu_t_hbm.at[:, pl.ds(ubase, _IDS_U)])


_BLK_U = 512


def _mm_body(wu_t_ref, hi_t_ref, out_ref):
    out_ref[...] = lax.dot_general(
        wu_t_ref[...], hi_t_ref[...],
        dimension_numbers=(((0,), (0,)), ((), ())),
        preferred_element_type=jnp.float32,
    )


_matmul = pl.pallas_call(
    _mm_body,
    grid=(BATCH_U // _BLK_U,),
    in_specs=[
        pl.BlockSpec((NUM_COMPONENTS, _BLK_U), lambda i: (0, i)),
        pl.BlockSpec((NUM_COMPONENTS, BATCH_I), lambda i: (0, 0)),
    ],
    out_specs=pl.BlockSpec((_BLK_U, BATCH_I), lambda i: (i, 0)),
    out_shape=jax.ShapeDtypeStruct((BATCH_U, BATCH_I), jnp.float32),
)


@jax.jit
def kernel(user_tensor, item_tensor, user_embedding, item_embedding):
    ue_t = user_embedding.T
    ie_t = item_embedding.T
    hi_t, wu_t = _sc_gather(item_tensor, user_tensor, ie_t, ue_t)
    return _matmul(wu_t, hi_t)
